# Initial kernel scaffold; baseline (speedup 1.0000x reference)
import jax
import jax.numpy as jnp
from jax.experimental import pallas as pl


def kernel(boxes, scores):
    raise NotImplementedError("write your pallas kernel here")


# speedup vs baseline: 21.9855x; 21.9855x over previous
import jax
import jax.numpy as jnp
from jax import lax
from jax.experimental import pallas as pl

_N = 20000
_R = 160
_C = 128
_NPAD = _R * _C
_MAX_OUT = 100
_IOU_THRESHOLD = 0.5


def _nms_body(x1_ref, y1_ref, x2_ref, y2_ref, s_ref, out_ref):
    x1 = x1_ref[...]
    y1 = y1_ref[...]
    x2 = x2_ref[...]
    y2 = y2_ref[...]
    s = s_ref[...]
    areas = jnp.maximum(x2 - x1, 0.0) * jnp.maximum(y2 - y1, 0.0)
    flat = (lax.broadcasted_iota(jnp.int32, (_R, _C), 0) * _C
            + lax.broadcasted_iota(jnp.int32, (_R, _C), 1))
    flat8 = (lax.broadcasted_iota(jnp.int32, (8, _C), 0) * _C
             + lax.broadcasted_iota(jnp.int32, (8, _C), 1))
    alive0 = jnp.where(flat < _N, 1.0, 0.0)
    zeros8 = jnp.zeros((8, _C), jnp.float32)

    def body(i, carry):
        alive, ox1, oy1, ox2, oy2, osc = carry
        ms = jnp.where(alive > 0.0, s, -2.0)
        m = jnp.max(ms)
        has = m > -1.0
        eq = ms == m
        idx = jnp.min(jnp.where(eq, flat, jnp.int32(2**30)))
        sel = flat == idx
        cx1 = jnp.sum(jnp.where(sel, x1, 0.0))
        cy1 = jnp.sum(jnp.where(sel, y1, 0.0))
        cx2 = jnp.sum(jnp.where(sel, x2, 0.0))
        cy2 = jnp.sum(jnp.where(sel, y2, 0.0))
        carea = jnp.sum(jnp.where(sel, areas, 0.0))
        xx1 = jnp.maximum(cx1, x1)
        yy1 = jnp.maximum(cy1, y1)
        xx2 = jnp.minimum(cx2, x2)
        yy2 = jnp.minimum(cy2, y2)
        inter = jnp.maximum(xx2 - xx1, 0.0) * jnp.maximum(yy2 - yy1, 0.0)
        iou = inter / (carea + areas - inter + 1e-9)
        supp = jnp.logical_or(iou > _IOU_THRESHOLD, sel)
        alive = jnp.where(jnp.logical_and(has, supp), 0.0, alive)
        hf = jnp.where(has, 1.0, 0.0)
        upd = flat8 == i
        ox1 = jnp.where(upd, cx1 * hf, ox1)
        oy1 = jnp.where(upd, cy1 * hf, oy1)
        ox2 = jnp.where(upd, cx2 * hf, ox2)
        oy2 = jnp.where(upd, cy2 * hf, oy2)
        osc = jnp.where(upd, m * hf, osc)
        return alive, ox1, oy1, ox2, oy2, osc

    init = (alive0, zeros8, zeros8, zeros8, zeros8, zeros8)
    _, ox1, oy1, ox2, oy2, osc = lax.fori_loop(0, _MAX_OUT, body, init)
    out_ref[0:8, :] = ox1
    out_ref[8:16, :] = oy1
    out_ref[16:24, :] = ox2
    out_ref[24:32, :] = oy2
    out_ref[32:40, :] = osc


def kernel(boxes, scores):
    bp = jnp.pad(boxes, ((0, _NPAD - _N), (0, 0)))
    sp = jnp.pad(scores, (0, _NPAD - _N)).reshape(_R, _C)
    planes = [bp[:, k].reshape(_R, _C) for k in range(4)]
    out = pl.pallas_call(
        _nms_body,
        out_shape=jax.ShapeDtypeStruct((40, _C), jnp.float32),
    )(planes[0], planes[1], planes[2], planes[3], sp)
    g = out.reshape(5, 8 * _C)[:, :_MAX_OUT]
    return g.T
